# baseline (speedup 1.0000x reference)
import functools

import jax
import jax.numpy as jnp
from jax import lax
from jax.experimental import pallas as pl
from jax.experimental.pallas import tpu as pltpu
from jax.experimental.pallas import tpu_sc as plsc

_NW = 32
_L = 16
_EPB = 120
_NBLK = 41


def _project_body(m_ref, b_ref, o_ref):
    o_ref[...] = jnp.dot(
        m_ref[...], b_ref[...], preferred_element_type=jnp.float32)


def _project(mesh_signal, blog, n, blk_m):
    nc = blog.shape[1]
    return pl.pallas_call(
        _project_body,
        grid=(pl.cdiv(n, blk_m),),
        in_specs=[
            pl.BlockSpec((blk_m, mesh_signal.shape[1]), lambda i: (i, 0)),
            pl.BlockSpec(blog.shape, lambda i: (0, 0)),
        ],
        out_specs=pl.BlockSpec((blk_m, nc), lambda i: (i, 0)),
        out_shape=jax.ShapeDtypeStruct((n, nc), jnp.float32),
    )(mesh_signal, blog)


def _make_sc_kernel(n, nv_t):
    ne_t = nv_t * _EPB
    mesh = plsc.VectorSubcoreMesh(
        core_axis_name="c", subcore_axis_name="s",
        num_cores=2, num_subcores=16)

    ne2 = nv_t * 128

    @functools.partial(
        pl.kernel,
        out_type=jax.ShapeDtypeStruct((n * 32,), jnp.float32),
        mesh=mesh,
        compiler_params=pltpu.CompilerParams(
            needs_layout_passes=False, use_tc_tiling_on_sc=False),
        scratch_types=[
            pltpu.VMEM((ne2,), jnp.int32),
            pltpu.VMEM((ne2,), jnp.float32),
            pltpu.VMEM((128, 32), jnp.float32),
            pltpu.VMEM((128, 32), jnp.float32),
            pltpu.VMEM((32,), jnp.float32),
            pltpu.VMEM((nv_t * 32,), jnp.float32),
            pltpu.SemaphoreType.DMA,
            pltpu.SemaphoreType.DMA,
        ],
    )
    def sc_kernel(tab, flath, wh, biash, out,
                  flatb, wsb, g_a, g_b, biasb, outb, sem_a, sem_b):
        wid = lax.axis_index("s") * 2 + lax.axis_index("c")
        g0 = lax.min(wid * nv_t, n - nv_t)

        pltpu.sync_copy(flath.at[pl.ds(g0 * 128, ne2)], flatb)
        pltpu.sync_copy(wh.at[pl.ds(g0 * 128, ne2)], wsb)
        pltpu.sync_copy(biash, biasb)

        bias_a = biasb[pl.ds(0, _L)]
        bias_b = biasb[pl.ds(_L, _L)]
        zero = jnp.zeros((_L,), jnp.float32)

        def fire(vl, gbuf, sem):
            pltpu.async_copy(
                tab.at[flatb.at[pl.ds(vl * 128, 128)]], gbuf, sem)

        def wait(vl, gbuf, sem):
            pltpu.make_async_copy(
                tab.at[flatb.at[pl.ds(vl * 128, 128)]], gbuf, sem).wait()

        def accumulate(vl, gbuf):
            base = vl * 128

            def acc_body(j, carry):
                accs = list(carry)
                wvec = wsb[pl.ds(base + 16 * j, _L)]
                for u in range(16):
                    e = j * 16 + u
                    wv = lax.broadcast(wvec[u], (_L,))
                    r0 = gbuf[e, pl.ds(0, _L)]
                    r1 = gbuf[e, pl.ds(_L, _L)]
                    k = 2 * (u % 4)
                    accs[k] = accs[k] + wv * r0
                    accs[k + 1] = accs[k + 1] + wv * r1
                return tuple(accs)

            init = (bias_a, bias_b) + (zero,) * 6
            accs = lax.fori_loop(0, 8, acc_body, init)
            acc_a = (accs[0] + accs[2]) + (accs[4] + accs[6])
            acc_b = (accs[1] + accs[3]) + (accs[5] + accs[7])
            outb[pl.ds(vl * 32, _L)] = jnp.maximum(acc_a, zero)
            outb[pl.ds(vl * 32 + _L, _L)] = jnp.maximum(acc_b, zero)

        fire(0, g_a, sem_a)

        def pair_body(v2, _):
            vl = v2 * 2
            fire(vl + 1, g_b, sem_b)
            wait(vl, g_a, sem_a)
            accumulate(vl, g_a)

            @pl.when(v2 < nv_t // 2 - 1)
            def _():
                fire(vl + 2, g_a, sem_a)

            wait(vl + 1, g_b, sem_b)
            accumulate(vl + 1, g_b)
            return 0

        lax.fori_loop(0, nv_t // 2, pair_body, 0)

        pltpu.sync_copy(outb, out.at[pl.ds(g0 * 32, nv_t * 32)])

    return sc_kernel


def _prep(mesh_signal, bary_coordinates, neighbor_weights, self_weights,
          bias):
    n, f = mesh_signal.shape
    t, r, a, _ = neighbor_weights.shape
    nj = a // 2
    nra = r * a
    assert nra * 3 == _EPB and nj * t == 32 and nra + 1 == _NBLK

    nv_t = 4 * ((n + 4 * _NW - 1) // (4 * _NW))

    wrot = jnp.stack(
        [jnp.roll(neighbor_weights, -2 * j, axis=2) for j in range(nj)],
        axis=0)
    bn = wrot.transpose(4, 2, 3, 0, 1).reshape(f, nra, nj * t)
    bc = jnp.tile(self_weights[:, 0, :], (nj, 1)).T[:, None, :]
    blog = jnp.concatenate([bn, bc], axis=1).reshape(f, _NBLK * 32)
    bias32 = jnp.tile(bias, (nj,))

    pat = jnp.repeat(jnp.arange(nra, dtype=jnp.int32), 3)
    flat2 = bary_coordinates[..., 0].astype(jnp.int32).reshape(n, _EPB)
    flat2 = flat2 * _NBLK + pat[None, :]
    center = (jnp.arange(n, dtype=jnp.int32) * _NBLK + (_NBLK - 1))[:, None]
    flatx = jnp.concatenate(
        [flat2, center, jnp.zeros((n, 7), jnp.int32)], axis=1)
    flatx = flatx.reshape(n * 128)
    wsx = jnp.concatenate(
        [bary_coordinates[..., 1].reshape(n, _EPB),
         jnp.ones((n, 1), jnp.float32),
         jnp.zeros((n, 7), jnp.float32)], axis=1).reshape(n * 128)
    return blog, bias32, flatx, wsx, n, nj, t, nv_t


def kernel(mesh_signal, bary_coordinates, neighbor_weights, self_weights,
           bias):
    blog, bias32, flatx, wsx, n, nj, t, nv_t = _prep(
        mesh_signal, bary_coordinates, neighbor_weights, self_weights, bias)

    p = _project(mesh_signal, blog, n, 1000)
    tab = p.reshape(n * _NBLK, nj * t)

    sck = _make_sc_kernel(n, nv_t)
    out_flat = sck(tab, flatx, wsx, bias32)

    return out_flat.reshape(n, nj, t)


# speedup vs baseline: 1.0030x; 1.0030x over previous
import functools

import jax
import jax.numpy as jnp
from jax import lax
from jax.experimental import pallas as pl
from jax.experimental.pallas import tpu as pltpu
from jax.experimental.pallas import tpu_sc as plsc

_NW = 32
_L = 16
_EPB = 120
_NBLK = 41


def _project_body(m_ref, b_ref, o_ref):
    o_ref[...] = jnp.dot(
        m_ref[...], b_ref[...], preferred_element_type=jnp.float32)


def _project(mesh_signal, blog, n, blk_m):
    nc = blog.shape[1]
    return pl.pallas_call(
        _project_body,
        grid=(pl.cdiv(n, blk_m),),
        in_specs=[
            pl.BlockSpec((blk_m, mesh_signal.shape[1]), lambda i: (i, 0)),
            pl.BlockSpec(blog.shape, lambda i: (0, 0)),
        ],
        out_specs=pl.BlockSpec((blk_m, nc), lambda i: (i, 0)),
        out_shape=jax.ShapeDtypeStruct((n, nc), jnp.float32),
    )(mesh_signal, blog)


def _make_sc_kernel(n, nv_t):
    ne_t = nv_t * _EPB
    mesh = plsc.VectorSubcoreMesh(
        core_axis_name="c", subcore_axis_name="s",
        num_cores=2, num_subcores=16)

    ne2 = nv_t * 128

    @functools.partial(
        pl.kernel,
        out_type=jax.ShapeDtypeStruct((n * 32,), jnp.float32),
        mesh=mesh,
        compiler_params=pltpu.CompilerParams(
            needs_layout_passes=False, use_tc_tiling_on_sc=False),
        scratch_types=[
            pltpu.VMEM((ne2,), jnp.int32),
            pltpu.VMEM((ne2,), jnp.float32),
            pltpu.VMEM((128, 32), jnp.float32),
            pltpu.VMEM((128, 32), jnp.float32),
            pltpu.VMEM((32,), jnp.float32),
            pltpu.VMEM((nv_t * 32,), jnp.float32),
            pltpu.SemaphoreType.DMA,
            pltpu.SemaphoreType.DMA,
        ],
    )
    def sc_kernel(tab, flath, wh, biash, out,
                  flatb, wsb, g_a, g_b, biasb, outb, sem_a, sem_b):
        wid = lax.axis_index("s") * 2 + lax.axis_index("c")
        g0 = lax.min(wid * nv_t, n - nv_t)

        pltpu.sync_copy(flath.at[pl.ds(g0 * 128, ne2)], flatb)
        pltpu.sync_copy(wh.at[pl.ds(g0 * 128, ne2)], wsb)
        pltpu.sync_copy(biash, biasb)

        bias_a = biasb[pl.ds(0, _L)]
        bias_b = biasb[pl.ds(_L, _L)]
        zero = jnp.zeros((_L,), jnp.float32)

        def fire(vl, gbuf, sem):
            pltpu.async_copy(
                tab.at[flatb.at[pl.ds(vl * 128, 128)]], gbuf, sem)

        def wait(vl, gbuf, sem):
            pltpu.make_async_copy(
                tab.at[flatb.at[pl.ds(vl * 128, 128)]], gbuf, sem).wait()

        def accumulate(vl, gbuf):
            base = vl * 128

            def acc_body(j, carry):
                accs = list(carry)
                wb = lax.broadcast(base + 8 * j, (_L,))
                for u in range(8):
                    e = j * 8 + u
                    wv = plsc.load_gather(wsb, [wb + u])
                    r0 = gbuf[e, pl.ds(0, _L)]
                    r1 = gbuf[e, pl.ds(_L, _L)]
                    k = 2 * (u % 4)
                    accs[k] = accs[k] + wv * r0
                    accs[k + 1] = accs[k + 1] + wv * r1
                return tuple(accs)

            init = (bias_a, bias_b) + (zero,) * 6
            accs = lax.fori_loop(0, 16, acc_body, init)
            acc_a = (accs[0] + accs[2]) + (accs[4] + accs[6])
            acc_b = (accs[1] + accs[3]) + (accs[5] + accs[7])
            outb[pl.ds(vl * 32, _L)] = jnp.maximum(acc_a, zero)
            outb[pl.ds(vl * 32 + _L, _L)] = jnp.maximum(acc_b, zero)

        fire(0, g_a, sem_a)

        def pair_body(v2, _):
            vl = v2 * 2
            fire(vl + 1, g_b, sem_b)
            wait(vl, g_a, sem_a)
            accumulate(vl, g_a)

            @pl.when(v2 < nv_t // 2 - 1)
            def _():
                fire(vl + 2, g_a, sem_a)

            wait(vl + 1, g_b, sem_b)
            accumulate(vl + 1, g_b)
            return 0

        lax.fori_loop(0, nv_t // 2, pair_body, 0)

        pltpu.sync_copy(outb, out.at[pl.ds(g0 * 32, nv_t * 32)])

    return sc_kernel


def _prep(mesh_signal, bary_coordinates, neighbor_weights, self_weights,
          bias):
    n, f = mesh_signal.shape
    t, r, a, _ = neighbor_weights.shape
    nj = a // 2
    nra = r * a
    assert nra * 3 == _EPB and nj * t == 32 and nra + 1 == _NBLK

    nv_t = 4 * ((n + 4 * _NW - 1) // (4 * _NW))

    wrot = jnp.stack(
        [jnp.roll(neighbor_weights, -2 * j, axis=2) for j in range(nj)],
        axis=0)
    bn = wrot.transpose(4, 2, 3, 0, 1).reshape(f, nra, nj * t)
    bc = jnp.tile(self_weights[:, 0, :], (nj, 1)).T[:, None, :]
    blog = jnp.concatenate([bn, bc], axis=1).reshape(f, _NBLK * 32)
    bias32 = jnp.tile(bias, (nj,))

    pat = jnp.repeat(jnp.arange(nra, dtype=jnp.int32), 3)
    flat2 = bary_coordinates[..., 0].astype(jnp.int32).reshape(n, _EPB)
    flat2 = flat2 * _NBLK + pat[None, :]
    center = (jnp.arange(n, dtype=jnp.int32) * _NBLK + (_NBLK - 1))[:, None]
    flatx = jnp.concatenate(
        [flat2, center, jnp.zeros((n, 7), jnp.int32)], axis=1)
    flatx = flatx.reshape(n * 128)
    wsx = jnp.concatenate(
        [bary_coordinates[..., 1].reshape(n, _EPB),
         jnp.ones((n, 1), jnp.float32),
         jnp.zeros((n, 7), jnp.float32)], axis=1).reshape(n * 128)
    return blog, bias32, flatx, wsx, n, nj, t, nv_t


def kernel(mesh_signal, bary_coordinates, neighbor_weights, self_weights,
           bias):
    blog, bias32, flatx, wsx, n, nj, t, nv_t = _prep(
        mesh_signal, bary_coordinates, neighbor_weights, self_weights, bias)

    p = _project(mesh_signal, blog, n, 1000)
    tab = p.reshape(n * _NBLK, nj * t)

    sck = _make_sc_kernel(n, nv_t)
    out_flat = sck(tab, flatx, wsx, bias32)

    return out_flat.reshape(n, nj, t)


# speedup vs baseline: 2.5350x; 2.5275x over previous
import functools

import jax
import jax.numpy as jnp
from jax import lax
from jax.experimental import pallas as pl
from jax.experimental.pallas import tpu as pltpu
from jax.experimental.pallas import tpu_sc as plsc

_NW = 32
_L = 16
_EPB = 120
_NBLK = 41


def _project_body(m_ref, b_ref, o_ref):
    o_ref[...] = jnp.dot(
        m_ref[...], b_ref[...], preferred_element_type=jnp.float32)


def _project(mesh_signal, blog, n, blk_m):
    nc = blog.shape[1]
    return pl.pallas_call(
        _project_body,
        grid=(pl.cdiv(n, blk_m),),
        in_specs=[
            pl.BlockSpec((blk_m, mesh_signal.shape[1]), lambda i: (i, 0)),
            pl.BlockSpec(blog.shape, lambda i: (0, 0)),
        ],
        out_specs=pl.BlockSpec((blk_m, nc), lambda i: (i, 0)),
        out_shape=jax.ShapeDtypeStruct((n, nc), jnp.float32),
    )(mesh_signal, blog)


def _make_sc_kernel(n, nv_t):
    ne_t = nv_t * _EPB
    mesh = plsc.VectorSubcoreMesh(
        core_axis_name="c", subcore_axis_name="s",
        num_cores=2, num_subcores=16)

    ne2 = nv_t * 128

    @functools.partial(
        pl.kernel,
        out_type=jax.ShapeDtypeStruct((n * 32,), jnp.float32),
        mesh=mesh,
        compiler_params=pltpu.CompilerParams(
            needs_layout_passes=False, use_tc_tiling_on_sc=False),
        scratch_types=[
            pltpu.VMEM((ne2,), jnp.int32),
            pltpu.VMEM((ne2,), jnp.float32),
            pltpu.VMEM((128, 32), jnp.float32),
            pltpu.VMEM((128, 32), jnp.float32),
            pltpu.VMEM((32,), jnp.float32),
            pltpu.VMEM((nv_t * 32,), jnp.float32),
            pltpu.SemaphoreType.DMA,
            pltpu.SemaphoreType.DMA,
        ],
    )
    def sc_kernel(tab, flath, wh, biash, out,
                  flatb, wsb, g_a, g_b, biasb, outb, sem_a, sem_b):
        wid = lax.axis_index("s") * 2 + lax.axis_index("c")
        g0 = lax.min(wid * nv_t, n - nv_t)

        pltpu.sync_copy(flath.at[pl.ds(g0 * 128, ne2)], flatb)
        pltpu.sync_copy(wh.at[pl.ds(g0 * 128, ne2)], wsb)
        pltpu.sync_copy(biash, biasb)

        bias_a = biasb[pl.ds(0, _L)]
        bias_b = biasb[pl.ds(_L, _L)]
        zero = jnp.zeros((_L,), jnp.float32)

        for gbuf0 in (g_a, g_b):
            for e0 in range(_EPB + 1, 128):
                gbuf0[e0, pl.ds(0, _L)] = jnp.zeros((_L,), jnp.float32)
                gbuf0[e0, pl.ds(_L, _L)] = jnp.zeros((_L,), jnp.float32)

        def fire(vl, gbuf, sem):
            pltpu.async_copy(
                tab.at[flatb.at[pl.ds(vl * 128, _EPB + 1)]],
                gbuf.at[pl.ds(0, _EPB + 1)], sem)

        def wait(vl, gbuf, sem):
            pltpu.make_async_copy(
                tab.at[flatb.at[pl.ds(vl * 128, _EPB + 1)]],
                gbuf.at[pl.ds(0, _EPB + 1)], sem).wait()

        def accumulate(vl, gbuf):
            base = vl * 128

            def acc_body(j, carry):
                accs = list(carry)
                wb = lax.broadcast(base + 8 * j, (_L,))
                for u in range(8):
                    e = j * 8 + u
                    wv = plsc.load_gather(wsb, [wb + u])
                    r0 = gbuf[e, pl.ds(0, _L)]
                    r1 = gbuf[e, pl.ds(_L, _L)]
                    k = 2 * (u % 4)
                    accs[k] = accs[k] + wv * r0
                    accs[k + 1] = accs[k + 1] + wv * r1
                return tuple(accs)

            init = (bias_a, bias_b) + (zero,) * 6
            accs = lax.fori_loop(0, 16, acc_body, init)
            acc_a = (accs[0] + accs[2]) + (accs[4] + accs[6])
            acc_b = (accs[1] + accs[3]) + (accs[5] + accs[7])
            outb[pl.ds(vl * 32, _L)] = jnp.maximum(acc_a, zero)
            outb[pl.ds(vl * 32 + _L, _L)] = jnp.maximum(acc_b, zero)

        fire(0, g_a, sem_a)

        def pair_body(v2, _):
            vl = v2 * 2
            fire(vl + 1, g_b, sem_b)
            wait(vl, g_a, sem_a)
            accumulate(vl, g_a)

            @pl.when(v2 < nv_t // 2 - 1)
            def _():
                fire(vl + 2, g_a, sem_a)

            wait(vl + 1, g_b, sem_b)
            accumulate(vl + 1, g_b)
            return 0

        lax.fori_loop(0, nv_t // 2, pair_body, 0)

        pltpu.sync_copy(outb, out.at[pl.ds(g0 * 32, nv_t * 32)])

    return sc_kernel


def _prep(mesh_signal, bary_coordinates, neighbor_weights, self_weights,
          bias):
    n, f = mesh_signal.shape
    t, r, a, _ = neighbor_weights.shape
    nj = a // 2
    nra = r * a
    assert nra * 3 == _EPB and nj * t == 32 and nra + 1 == _NBLK

    nv_t = 4 * ((n + 4 * _NW - 1) // (4 * _NW))

    wrot = jnp.stack(
        [jnp.roll(neighbor_weights, -2 * j, axis=2) for j in range(nj)],
        axis=0)
    bn = wrot.transpose(4, 2, 3, 0, 1).reshape(f, nra, nj * t)
    bc = jnp.tile(self_weights[:, 0, :], (nj, 1)).T[:, None, :]
    blog = jnp.concatenate([bn, bc], axis=1).reshape(f, _NBLK * 32)
    bias32 = jnp.tile(bias, (nj,))

    pat = jnp.repeat(jnp.arange(nra, dtype=jnp.int32), 3)
    flat2 = bary_coordinates[..., 0].astype(jnp.int32).reshape(n, _EPB)
    flat2 = flat2 * _NBLK + pat[None, :]
    center = (jnp.arange(n, dtype=jnp.int32) * _NBLK + (_NBLK - 1))[:, None]
    flatx = jnp.concatenate(
        [flat2, center, jnp.zeros((n, 7), jnp.int32)], axis=1)
    flatx = flatx.reshape(n * 128)
    wsx = jnp.concatenate(
        [bary_coordinates[..., 1].reshape(n, _EPB),
         jnp.ones((n, 1), jnp.float32),
         jnp.zeros((n, 7), jnp.float32)], axis=1).reshape(n * 128)
    return blog, bias32, flatx, wsx, n, nj, t, nv_t


def kernel(mesh_signal, bary_coordinates, neighbor_weights, self_weights,
           bias):
    blog, bias32, flatx, wsx, n, nj, t, nv_t = _prep(
        mesh_signal, bary_coordinates, neighbor_weights, self_weights, bias)

    p = _project(mesh_signal, blog, n, 1000)
    tab = p.reshape(n * _NBLK, nj * t)

    sck = _make_sc_kernel(n, nv_t)
    out_flat = sck(tab, flatx, wsx, bias32)

    return out_flat.reshape(n, nj, t)


# speedup vs baseline: 3.1324x; 1.2357x over previous
import functools

import jax
import jax.numpy as jnp
from jax import lax
from jax.experimental import pallas as pl
from jax.experimental.pallas import tpu as pltpu
from jax.experimental.pallas import tpu_sc as plsc

_NW = 32
_L = 16
_EPB = 120
_NBLK = 41


def _project_body(m_ref, b_ref, o_ref):
    o_ref[...] = jnp.dot(
        m_ref[...], b_ref[...], preferred_element_type=jnp.float32)


def _project(mesh_signal, blog, n, blk_m):
    nc = blog.shape[1]
    return pl.pallas_call(
        _project_body,
        grid=(pl.cdiv(n, blk_m),),
        in_specs=[
            pl.BlockSpec((blk_m, mesh_signal.shape[1]), lambda i: (i, 0)),
            pl.BlockSpec(blog.shape, lambda i: (0, 0)),
        ],
        out_specs=pl.BlockSpec((blk_m, nc), lambda i: (i, 0)),
        out_shape=jax.ShapeDtypeStruct((n, nc), jnp.float32),
    )(mesh_signal, blog)


def _make_sc_kernel(n, nv_t):
    ne_t = nv_t * _EPB
    mesh = plsc.VectorSubcoreMesh(
        core_axis_name="c", subcore_axis_name="s",
        num_cores=2, num_subcores=16)

    ne2 = nv_t * 128

    @functools.partial(
        pl.kernel,
        out_type=jax.ShapeDtypeStruct((n * 32,), jnp.float32),
        mesh=mesh,
        compiler_params=pltpu.CompilerParams(
            needs_layout_passes=False, use_tc_tiling_on_sc=False),
        scratch_types=[
            pltpu.VMEM((ne2,), jnp.int32),
            pltpu.VMEM((ne2,), jnp.float32),
            pltpu.VMEM((128, 32), jnp.float32),
            pltpu.VMEM((128, 32), jnp.float32),
            pltpu.VMEM((128, 32), jnp.float32),
            pltpu.VMEM((128, 32), jnp.float32),
            pltpu.VMEM((32,), jnp.float32),
            pltpu.VMEM((nv_t * 32,), jnp.float32),
            pltpu.SemaphoreType.DMA,
            pltpu.SemaphoreType.DMA,
            pltpu.SemaphoreType.DMA,
            pltpu.SemaphoreType.DMA,
        ],
    )
    def sc_kernel(tab, flath, wh, biash, out,
                  flatb, wsb, g_a, g_b, g_c, g_d, biasb, outb,
                  sem_a, sem_b, sem_c, sem_d):
        wid = lax.axis_index("s") * 2 + lax.axis_index("c")
        g0 = lax.min(wid * nv_t, n - nv_t)

        pltpu.sync_copy(flath.at[pl.ds(g0 * 128, ne2)], flatb)
        pltpu.sync_copy(wh.at[pl.ds(g0 * 128, ne2)], wsb)
        pltpu.sync_copy(biash, biasb)

        bias_a = biasb[pl.ds(0, _L)]
        bias_b = biasb[pl.ds(_L, _L)]
        zero = jnp.zeros((_L,), jnp.float32)

        for gbuf0 in (g_a, g_b, g_c, g_d):
            for e0 in range(_EPB + 1, 128):
                gbuf0[e0, pl.ds(0, _L)] = jnp.zeros((_L,), jnp.float32)
                gbuf0[e0, pl.ds(_L, _L)] = jnp.zeros((_L,), jnp.float32)

        def fire(vl, gbuf, sem):
            pltpu.async_copy(
                tab.at[flatb.at[pl.ds(vl * 128, _EPB + 1)]],
                gbuf.at[pl.ds(0, _EPB + 1)], sem)

        def wait(vl, gbuf, sem):
            pltpu.make_async_copy(
                tab.at[flatb.at[pl.ds(vl * 128, _EPB + 1)]],
                gbuf.at[pl.ds(0, _EPB + 1)], sem).wait()

        def accumulate(vl, gbuf):
            base = vl * 128

            def acc_body(j, carry):
                accs = list(carry)
                wb = lax.broadcast(base + 8 * j, (_L,))
                for u in range(8):
                    e = j * 8 + u
                    wv = plsc.load_gather(wsb, [wb + u])
                    r0 = gbuf[e, pl.ds(0, _L)]
                    r1 = gbuf[e, pl.ds(_L, _L)]
                    k = 2 * (u % 4)
                    accs[k] = accs[k] + wv * r0
                    accs[k + 1] = accs[k + 1] + wv * r1
                return tuple(accs)

            init = (bias_a, bias_b) + (zero,) * 6
            accs = lax.fori_loop(0, 16, acc_body, init)
            acc_a = (accs[0] + accs[2]) + (accs[4] + accs[6])
            acc_b = (accs[1] + accs[3]) + (accs[5] + accs[7])
            outb[pl.ds(vl * 32, _L)] = jnp.maximum(acc_a, zero)
            outb[pl.ds(vl * 32 + _L, _L)] = jnp.maximum(acc_b, zero)

        bufs = ((g_a, sem_a), (g_b, sem_b), (g_c, sem_c), (g_d, sem_d))
        for b in range(3):
            fire(b, *bufs[b])

        def quad_body(v4, _):
            vl = v4 * 4
            for b in range(4):
                nxt = vl + b + 3

                @pl.when(nxt < nv_t)
                def _():
                    fire(nxt, *bufs[(b + 3) % 4])

                wait(vl + b, *bufs[b])
                accumulate(vl + b, bufs[b][0])
            return 0

        lax.fori_loop(0, nv_t // 4, quad_body, 0)

        pltpu.sync_copy(outb, out.at[pl.ds(g0 * 32, nv_t * 32)])

    return sc_kernel


def _prep(mesh_signal, bary_coordinates, neighbor_weights, self_weights,
          bias):
    n, f = mesh_signal.shape
    t, r, a, _ = neighbor_weights.shape
    nj = a // 2
    nra = r * a
    assert nra * 3 == _EPB and nj * t == 32 and nra + 1 == _NBLK

    nv_t = 4 * ((n + 4 * _NW - 1) // (4 * _NW))

    wrot = jnp.stack(
        [jnp.roll(neighbor_weights, -2 * j, axis=2) for j in range(nj)],
        axis=0)
    bn = wrot.transpose(4, 2, 3, 0, 1).reshape(f, nra, nj * t)
    bc = jnp.tile(self_weights[:, 0, :], (nj, 1)).T[:, None, :]
    blog = jnp.concatenate([bn, bc], axis=1).reshape(f, _NBLK * 32)
    bias32 = jnp.tile(bias, (nj,))

    pat = jnp.repeat(jnp.arange(nra, dtype=jnp.int32), 3)
    flat2 = bary_coordinates[..., 0].astype(jnp.int32).reshape(n, _EPB)
    flat2 = flat2 * _NBLK + pat[None, :]
    center = (jnp.arange(n, dtype=jnp.int32) * _NBLK + (_NBLK - 1))[:, None]
    flatx = jnp.concatenate(
        [flat2, center, jnp.zeros((n, 7), jnp.int32)], axis=1)
    flatx = flatx.reshape(n * 128)
    wsx = jnp.concatenate(
        [bary_coordinates[..., 1].reshape(n, _EPB),
         jnp.ones((n, 1), jnp.float32),
         jnp.zeros((n, 7), jnp.float32)], axis=1).reshape(n * 128)
    return blog, bias32, flatx, wsx, n, nj, t, nv_t


def kernel(mesh_signal, bary_coordinates, neighbor_weights, self_weights,
           bias):
    blog, bias32, flatx, wsx, n, nj, t, nv_t = _prep(
        mesh_signal, bary_coordinates, neighbor_weights, self_weights, bias)

    p = _project(mesh_signal, blog, n, 1000)
    tab = p.reshape(n * _NBLK, nj * t)

    sck = _make_sc_kernel(n, nv_t)
    out_flat = sck(tab, flatx, wsx, bias32)

    return out_flat.reshape(n, nj, t)


# speedup vs baseline: 3.2488x; 1.0372x over previous
import functools

import jax
import jax.numpy as jnp
from jax import lax
from jax.experimental import pallas as pl
from jax.experimental.pallas import tpu as pltpu
from jax.experimental.pallas import tpu_sc as plsc

_NW = 32
_L = 16
_EPB = 120
_NBLK = 41


def _project_body(m_ref, b_ref, o_ref):
    o_ref[...] = jnp.dot(
        m_ref[...], b_ref[...], preferred_element_type=jnp.float32)


def _project(mesh_signal, blog, n, blk_m):
    nc = blog.shape[1]
    return pl.pallas_call(
        _project_body,
        grid=(pl.cdiv(n, blk_m),),
        in_specs=[
            pl.BlockSpec((blk_m, mesh_signal.shape[1]), lambda i: (i, 0)),
            pl.BlockSpec(blog.shape, lambda i: (0, 0)),
        ],
        out_specs=pl.BlockSpec((blk_m, nc), lambda i: (i, 0)),
        out_shape=jax.ShapeDtypeStruct((n, nc), jnp.float32),
    )(mesh_signal, blog)


def _make_sc_kernel(n, nv_t):
    ne_t = nv_t * _EPB
    mesh = plsc.VectorSubcoreMesh(
        core_axis_name="c", subcore_axis_name="s",
        num_cores=2, num_subcores=16)

    ne2 = nv_t * 128

    @functools.partial(
        pl.kernel,
        out_type=jax.ShapeDtypeStruct((n * 32,), jnp.float32),
        mesh=mesh,
        compiler_params=pltpu.CompilerParams(
            needs_layout_passes=False, use_tc_tiling_on_sc=False),
        scratch_types=[
            pltpu.VMEM((ne2,), jnp.int32),
            pltpu.VMEM((ne2,), jnp.float32),
            pltpu.VMEM((128, 32), jnp.float32),
            pltpu.VMEM((128, 32), jnp.float32),
            pltpu.VMEM((128, 32), jnp.float32),
            pltpu.VMEM((128, 32), jnp.float32),
            pltpu.VMEM((128, 32), jnp.float32),
            pltpu.VMEM((128, 32), jnp.float32),
            pltpu.VMEM((32,), jnp.float32),
            pltpu.VMEM((nv_t * 32,), jnp.float32),
            pltpu.SemaphoreType.DMA,
            pltpu.SemaphoreType.DMA,
            pltpu.SemaphoreType.DMA,
            pltpu.SemaphoreType.DMA,
            pltpu.SemaphoreType.DMA,
            pltpu.SemaphoreType.DMA,
        ],
    )
    def sc_kernel(tab, flath, wh, biash, out,
                  flatb, wsb, g_a, g_b, g_c, g_d, g_e, g_f, biasb, outb,
                  sem_a, sem_b, sem_c, sem_d, sem_e, sem_f):
        wid = lax.axis_index("s") * 2 + lax.axis_index("c")
        g0 = lax.min(wid * nv_t, n - nv_t)

        pltpu.sync_copy(flath.at[pl.ds(g0 * 128, ne2)], flatb)
        pltpu.sync_copy(wh.at[pl.ds(g0 * 128, ne2)], wsb)
        pltpu.sync_copy(biash, biasb)

        bias_a = biasb[pl.ds(0, _L)]
        bias_b = biasb[pl.ds(_L, _L)]
        zero = jnp.zeros((_L,), jnp.float32)

        for gbuf0 in (g_a, g_b, g_c, g_d, g_e, g_f):
            for e0 in range(_EPB + 1, 128):
                gbuf0[e0, pl.ds(0, _L)] = jnp.zeros((_L,), jnp.float32)
                gbuf0[e0, pl.ds(_L, _L)] = jnp.zeros((_L,), jnp.float32)

        def fire(vl, gbuf, sem):
            pltpu.async_copy(
                tab.at[flatb.at[pl.ds(vl * 128, _EPB + 1)]],
                gbuf.at[pl.ds(0, _EPB + 1)], sem)

        def wait(vl, gbuf, sem):
            pltpu.make_async_copy(
                tab.at[flatb.at[pl.ds(vl * 128, _EPB + 1)]],
                gbuf.at[pl.ds(0, _EPB + 1)], sem).wait()

        def accumulate(vl, gbuf):
            base = vl * 128

            def acc_body(j, carry):
                accs = list(carry)
                wb = lax.broadcast(base + 8 * j, (_L,))
                for u in range(8):
                    e = j * 8 + u
                    wv = plsc.load_gather(wsb, [wb + u])
                    r0 = gbuf[e, pl.ds(0, _L)]
                    r1 = gbuf[e, pl.ds(_L, _L)]
                    k = 2 * (u % 4)
                    accs[k] = accs[k] + wv * r0
                    accs[k + 1] = accs[k + 1] + wv * r1
                return tuple(accs)

            init = (bias_a, bias_b) + (zero,) * 6
            accs = lax.fori_loop(0, 16, acc_body, init)
            acc_a = (accs[0] + accs[2]) + (accs[4] + accs[6])
            acc_b = (accs[1] + accs[3]) + (accs[5] + accs[7])
            outb[pl.ds(vl * 32, _L)] = jnp.maximum(acc_a, zero)
            outb[pl.ds(vl * 32 + _L, _L)] = jnp.maximum(acc_b, zero)

        bufs = ((g_a, sem_a), (g_b, sem_b), (g_c, sem_c), (g_d, sem_d),
                (g_e, sem_e), (g_f, sem_f))
        nb = len(bufs)
        for b in range(nb - 1):
            fire(b, *bufs[b])

        def round_body(vr, _):
            vl = vr * nb
            for b in range(nb):
                nxt = vl + b + nb - 1

                @pl.when(nxt < nv_t)
                def _():
                    fire(nxt, *bufs[(b + nb - 1) % nb])

                wait(vl + b, *bufs[b])
                accumulate(vl + b, bufs[b][0])
            return 0

        lax.fori_loop(0, nv_t // nb, round_body, 0)

        for b in range(nv_t % nb):
            vl = (nv_t // nb) * nb + b
            wait(vl, *bufs[b])
            accumulate(vl, bufs[b][0])

        pltpu.sync_copy(outb, out.at[pl.ds(g0 * 32, nv_t * 32)])

    return sc_kernel


def _prep(mesh_signal, bary_coordinates, neighbor_weights, self_weights,
          bias):
    n, f = mesh_signal.shape
    t, r, a, _ = neighbor_weights.shape
    nj = a // 2
    nra = r * a
    assert nra * 3 == _EPB and nj * t == 32 and nra + 1 == _NBLK

    nv_t = 4 * ((n + 4 * _NW - 1) // (4 * _NW))

    wrot = jnp.stack(
        [jnp.roll(neighbor_weights, -2 * j, axis=2) for j in range(nj)],
        axis=0)
    bn = wrot.transpose(4, 2, 3, 0, 1).reshape(f, nra, nj * t)
    bc = jnp.tile(self_weights[:, 0, :], (nj, 1)).T[:, None, :]
    blog = jnp.concatenate([bn, bc], axis=1).reshape(f, _NBLK * 32)
    bias32 = jnp.tile(bias, (nj,))

    pat = jnp.repeat(jnp.arange(nra, dtype=jnp.int32), 3)
    flat2 = bary_coordinates[..., 0].astype(jnp.int32).reshape(n, _EPB)
    flat2 = flat2 * _NBLK + pat[None, :]
    center = (jnp.arange(n, dtype=jnp.int32) * _NBLK + (_NBLK - 1))[:, None]
    flatx = jnp.concatenate(
        [flat2, center, jnp.zeros((n, 7), jnp.int32)], axis=1)
    flatx = flatx.reshape(n * 128)
    wsx = jnp.concatenate(
        [bary_coordinates[..., 1].reshape(n, _EPB),
         jnp.ones((n, 1), jnp.float32),
         jnp.zeros((n, 7), jnp.float32)], axis=1).reshape(n * 128)
    return blog, bias32, flatx, wsx, n, nj, t, nv_t


def kernel(mesh_signal, bary_coordinates, neighbor_weights, self_weights,
           bias):
    blog, bias32, flatx, wsx, n, nj, t, nv_t = _prep(
        mesh_signal, bary_coordinates, neighbor_weights, self_weights, bias)

    p = _project(mesh_signal, blog, n, 1000)
    tab = p.reshape(n * _NBLK, nj * t)

    sck = _make_sc_kernel(n, nv_t)
    out_flat = sck(tab, flatx, wsx, bias32)

    return out_flat.reshape(n, nj, t)
